# baseline (speedup 1.0000x reference)
import functools

import jax
import jax.numpy as jnp
from jax import lax
from jax.experimental import pallas as pl
from jax.experimental.pallas import tpu as pltpu
from jax.experimental.pallas import tpu_sc as plsc

N = 10000
D = 128
DH = 64
E = 320000
K1 = 17
KP = 32
NC = 2
NS = 16
BLK = 128
NBLK_T = 160
NCHUNK = 16
NBLK = NS * NBLK_T
E_PAD = NBLK * BLK
NROW = 10240
RPT = NROW // NS
XPT = N // NS


def _sc_body(xs_h, src_h, dst_h, zz_h, out_h, src_v, dst_v, rows_a, rows_b,
             x_sh, acc_sh, sem_a, sem_b):
    c = lax.axis_index("c")
    s = lax.axis_index("s")
    pltpu.sync_copy(xs_h.at[c, pl.ds(s * XPT, XPT)], x_sh.at[pl.ds(s * XPT, XPT)])
    pltpu.sync_copy(zz_h.at[pl.ds(s * RPT, RPT)], acc_sh.at[pl.ds(s * RPT, RPT)])
    plsc.subcore_barrier()

    def chunk(sup, carry):
        base = s * NBLK_T + sup * NCHUNK
        pltpu.sync_copy(src_h.at[pl.ds(base, NCHUNK)], src_v)
        pltpu.sync_copy(dst_h.at[pl.ds(base, NCHUNK)], dst_v)
        pltpu.async_copy(x_sh.at[src_v.at[0]], rows_a, sem_a)

        def step(t, c2):
            j0 = 2 * t
            j1 = 2 * t + 1
            jn = jnp.minimum(j1 + 1, NCHUNK - 1)
            pltpu.make_async_copy(x_sh.at[src_v.at[j0]], rows_a, sem_a).wait()
            pltpu.async_copy(x_sh.at[src_v.at[j1]], rows_b, sem_b)
            pltpu.sync_copy(rows_a, acc_sh.at[dst_v.at[j0]], add=True)
            pltpu.make_async_copy(x_sh.at[src_v.at[j1]], rows_b, sem_b).wait()
            pltpu.async_copy(x_sh.at[src_v.at[jn]], rows_a, sem_a)
            pltpu.sync_copy(rows_b, acc_sh.at[dst_v.at[j1]], add=True)
            return c2

        lax.fori_loop(0, NCHUNK // 2, step, 0)
        pltpu.make_async_copy(x_sh.at[src_v.at[NCHUNK - 1]], rows_a, sem_a).wait()
        return carry

    lax.fori_loop(0, NBLK_T // NCHUNK, chunk, 0)
    plsc.subcore_barrier()
    pltpu.sync_copy(acc_sh.at[pl.ds(s * RPT, RPT)],
                    out_h.at[c, pl.ds(s * RPT, RPT)])


@functools.cache
def _sc_scatter():
    return pl.kernel(
        _sc_body,
        out_type=jax.ShapeDtypeStruct((NC, NROW, DH), jnp.float32),
        mesh=plsc.VectorSubcoreMesh(core_axis_name="c", subcore_axis_name="s",
                                    num_cores=NC, num_subcores=NS),
        scratch_types=[
            pltpu.VMEM((NCHUNK, BLK), jnp.int32),
            pltpu.VMEM((NCHUNK, BLK), jnp.int32),
            pltpu.VMEM((BLK, DH), jnp.float32),
            pltpu.VMEM((BLK, DH), jnp.float32),
            pltpu.VMEM_SHARED((N, DH), jnp.float32),
            pltpu.VMEM_SHARED((NROW, DH), jnp.float32),
            pltpu.SemaphoreType.DMA,
            pltpu.SemaphoreType.DMA,
        ],
        compiler_params=pltpu.CompilerParams(use_tc_tiling_on_sc=False),
    )


def _finish_body(p_ref, w_ref, s_ref, t_ref):
    agg = jnp.concatenate([p_ref[0, :N, :], p_ref[1, :N, :]], axis=1)
    sc = lax.dot_general(agg.astype(jnp.bfloat16), w_ref[...],
                         (((1,), (1,)), ((), ())),
                         preferred_element_type=jnp.float32)
    col = lax.broadcasted_iota(jnp.int32, (N, KP), 1)
    valid = col < K1
    sm = jnp.where(valid, sc, jnp.float32(-3.4e38))
    m = jnp.max(sm, axis=1, keepdims=True)
    hit = jnp.logical_and(sm == m, valid)
    idx = jnp.where(hit, col, jnp.int32(KP))
    t_ref[...] = jnp.min(idx, axis=1, keepdims=True)
    s_ref[...] = sc[:, :K1]


def _finish(partials, w_bf):
    return pl.pallas_call(
        _finish_body,
        out_shape=(jax.ShapeDtypeStruct((N, K1), jnp.float32),
                   jax.ShapeDtypeStruct((N, 1), jnp.int32)),
    )(partials, w_bf)


def kernel(x, edge_index, W_rnn, h0, a_prelu, W_dec):
    src = edge_index[0]
    dst = edge_index[1]
    pad = E_PAD - E
    src_p = jnp.concatenate([src, jnp.zeros((pad,), jnp.int32)]).reshape(NBLK, BLK)
    dst_p = jnp.concatenate([dst, jnp.full((pad,), N, jnp.int32)]).reshape(NBLK, BLK)
    zz = jnp.zeros((NROW, DH), jnp.float32)
    xs = jnp.stack([x[:, :DH], x[:, DH:]])

    def step(h, _):
        v = h @ W_rnn.T
        h_new = jnp.where(v >= 0, v, a_prelu * v)
        return h_new, h_new

    _, H = lax.scan(step, h0, None, length=K1)
    weights = H @ W_dec.T
    w_bf = jnp.concatenate(
        [weights, jnp.zeros((KP - K1, D), weights.dtype)]).astype(jnp.bfloat16)

    partials = _sc_scatter()(xs, src_p, dst_p, zz)
    scores, t = _finish(partials, w_bf)
    return scores, t.reshape(N)


# speedup vs baseline: 1.0789x; 1.0789x over previous
import functools

import jax
import jax.numpy as jnp
from jax import lax
from jax.experimental import pallas as pl
from jax.experimental.pallas import tpu as pltpu
from jax.experimental.pallas import tpu_sc as plsc

N = 10000
D = 128
DH = 64
E = 320000
K1 = 17
KP = 32
NC = 2
NS = 16
BLK = 128
NBLK_T = 160
NCHUNK = 16
NBLK = NS * NBLK_T
E_PAD = NBLK * BLK
NROW = 10240
RPT = NROW // NS
XPT = N // NS


def _sc_body(x_h, src_h, dst_h, zz_h, out_h, src_v, dst_v, rows_a, rows_b,
             x_sh, acc_sh, sem_a, sem_b):
    c = lax.axis_index("c")
    s = lax.axis_index("s")
    pltpu.sync_copy(x_h.at[pl.ds(s * XPT, XPT), pl.ds(c * DH, DH)],
                    x_sh.at[pl.ds(s * XPT, XPT)])
    pltpu.sync_copy(zz_h, rows_a)
    for q in range(RPT // BLK):
        pltpu.sync_copy(rows_a, acc_sh.at[pl.ds(s * RPT + q * BLK, BLK)])
    plsc.subcore_barrier()

    def chunk(sup, carry):
        base = s * NBLK_T + sup * NCHUNK
        pltpu.sync_copy(src_h.at[pl.ds(base, NCHUNK)], src_v)
        pltpu.sync_copy(dst_h.at[pl.ds(base, NCHUNK)], dst_v)
        pltpu.async_copy(x_sh.at[src_v.at[0]], rows_a, sem_a)

        def step(t, c2):
            j0 = 2 * t
            j1 = 2 * t + 1
            jn = jnp.minimum(j1 + 1, NCHUNK - 1)
            pltpu.make_async_copy(x_sh.at[src_v.at[j0]], rows_a, sem_a).wait()
            pltpu.async_copy(x_sh.at[src_v.at[j1]], rows_b, sem_b)
            pltpu.sync_copy(rows_a, acc_sh.at[dst_v.at[j0]], add=True)
            pltpu.make_async_copy(x_sh.at[src_v.at[j1]], rows_b, sem_b).wait()
            pltpu.async_copy(x_sh.at[src_v.at[jn]], rows_a, sem_a)
            pltpu.sync_copy(rows_b, acc_sh.at[dst_v.at[j1]], add=True)
            return c2

        lax.fori_loop(0, NCHUNK // 2, step, 0)
        pltpu.make_async_copy(x_sh.at[src_v.at[NCHUNK - 1]], rows_a, sem_a).wait()
        return carry

    lax.fori_loop(0, NBLK_T // NCHUNK, chunk, 0)
    plsc.subcore_barrier()
    pltpu.sync_copy(acc_sh.at[pl.ds(s * RPT, RPT)],
                    out_h.at[c, pl.ds(s * RPT, RPT)])


@functools.cache
def _sc_scatter():
    return pl.kernel(
        _sc_body,
        out_type=jax.ShapeDtypeStruct((NC, NROW, DH), jnp.float32),
        mesh=plsc.VectorSubcoreMesh(core_axis_name="c", subcore_axis_name="s",
                                    num_cores=NC, num_subcores=NS),
        scratch_types=[
            pltpu.VMEM((NCHUNK, BLK), jnp.int32),
            pltpu.VMEM((NCHUNK, BLK), jnp.int32),
            pltpu.VMEM((BLK, DH), jnp.float32),
            pltpu.VMEM((BLK, DH), jnp.float32),
            pltpu.VMEM_SHARED((N, DH), jnp.float32),
            pltpu.VMEM_SHARED((NROW, DH), jnp.float32),
            pltpu.SemaphoreType.DMA,
            pltpu.SemaphoreType.DMA,
        ],
        compiler_params=pltpu.CompilerParams(use_tc_tiling_on_sc=False),
    )


def _finish_body(p_ref, w_ref, s_ref, t_ref):
    agg = jnp.concatenate([p_ref[0, :N, :], p_ref[1, :N, :]], axis=1)
    sc = lax.dot_general(agg.astype(jnp.bfloat16), w_ref[...],
                         (((1,), (1,)), ((), ())),
                         preferred_element_type=jnp.float32)
    col = lax.broadcasted_iota(jnp.int32, (N, KP), 1)
    valid = col < K1
    sm = jnp.where(valid, sc, jnp.float32(-3.4e38))
    m = jnp.max(sm, axis=1, keepdims=True)
    hit = jnp.logical_and(sm == m, valid)
    idx = jnp.where(hit, col, jnp.int32(KP))
    t_ref[...] = jnp.min(idx, axis=1, keepdims=True)
    s_ref[...] = sc[:, :K1]


def _finish(partials, w_bf):
    return pl.pallas_call(
        _finish_body,
        out_shape=(jax.ShapeDtypeStruct((N, K1), jnp.float32),
                   jax.ShapeDtypeStruct((N, 1), jnp.int32)),
    )(partials, w_bf)


def kernel(x, edge_index, W_rnn, h0, a_prelu, W_dec):
    src = edge_index[0]
    dst = edge_index[1]
    pad = E_PAD - E
    src_p = jnp.concatenate([src, jnp.zeros((pad,), jnp.int32)]).reshape(NBLK, BLK)
    dst_p = jnp.concatenate([dst, jnp.full((pad,), N, jnp.int32)]).reshape(NBLK, BLK)
    zz = jnp.zeros((BLK, DH), jnp.float32)

    def step(h, _):
        v = h @ W_rnn.T
        h_new = jnp.where(v >= 0, v, a_prelu * v)
        return h_new, h_new

    _, H = lax.scan(step, h0, None, length=K1)
    weights = H @ W_dec.T
    w_bf = jnp.concatenate(
        [weights, jnp.zeros((KP - K1, D), weights.dtype)]).astype(jnp.bfloat16)

    partials = _sc_scatter()(x, src_p, dst_p, zz)
    scores, t = _finish(partials, w_bf)
    return scores, t.reshape(N)


# speedup vs baseline: 1.2224x; 1.1330x over previous
import functools

import jax
import jax.numpy as jnp
from jax import lax
from jax.experimental import pallas as pl
from jax.experimental.pallas import tpu as pltpu
from jax.experimental.pallas import tpu_sc as plsc

N = 10000
D = 128
DH = 64
E = 320000
K1 = 17
KP = 32
NC = 2
NS = 16
BLK = 128
NBLK_T = 160
NCHUNK = 16
NBLK = NS * NBLK_T
E_PAD = NBLK * BLK
NROW = 10240
RPT = NROW // NS
XPT = N // NS


def _sc_body(x_h, src_h, dst_h, zz_h, out_h, src_v, dst_v,
             rows_a, rows_b, rows_c, rows_d, x_sh, acc_sh,
             sem_a, sem_b, sem_c, sem_d, sem_e, sem_f, sem_g, sem_h):
    c = lax.axis_index("c")
    s = lax.axis_index("s")
    pltpu.sync_copy(x_h.at[pl.ds(s * XPT, XPT), pl.ds(c * DH, DH)],
                    x_sh.at[pl.ds(s * XPT, XPT)])
    pltpu.sync_copy(zz_h, rows_a)
    for q in range(RPT // BLK):
        pltpu.sync_copy(rows_a, acc_sh.at[pl.ds(s * RPT + q * BLK, BLK)])
    plsc.subcore_barrier()

    bufs = (rows_a, rows_b, rows_c, rows_d)
    gsems = (sem_a, sem_b, sem_c, sem_d)
    ssems = (sem_e, sem_f, sem_g, sem_h)

    def g_wait(k, j):
        pltpu.make_async_copy(x_sh.at[src_v.at[j]], bufs[k], gsems[k]).wait()

    def s_wait(k):
        pltpu.make_async_copy(bufs[k], acc_sh.at[dst_v.at[0]], ssems[k]).wait()

    def chunk(sup, carry):
        base = s * NBLK_T + sup * NCHUNK
        pltpu.sync_copy(src_h.at[pl.ds(base, NCHUNK)], src_v)
        pltpu.sync_copy(dst_h.at[pl.ds(base, NCHUNK)], dst_v)
        pltpu.async_copy(x_sh.at[src_v.at[0]], bufs[0], gsems[0])
        pltpu.async_copy(x_sh.at[src_v.at[1]], bufs[1], gsems[1])
        g_wait(0, 0)
        pltpu.async_copy(bufs[0], acc_sh.at[dst_v.at[0]], ssems[0], add=True)
        pltpu.async_copy(x_sh.at[src_v.at[2]], bufs[2], gsems[2])
        g_wait(1, 1)
        pltpu.async_copy(bufs[1], acc_sh.at[dst_v.at[1]], ssems[1], add=True)
        pltpu.async_copy(x_sh.at[src_v.at[3]], bufs[3], gsems[3])

        def step(t, c2):
            for k in range(4):
                j = 4 * t + 2 + k
                b = (2 + k) % 4
                nb = k
                g_wait(b, j)
                pltpu.async_copy(bufs[b], acc_sh.at[dst_v.at[j]],
                                 ssems[b], add=True)
                s_wait(nb)
                pltpu.async_copy(x_sh.at[src_v.at[j + 2]], bufs[nb], gsems[nb])
            return c2

        lax.fori_loop(0, (NCHUNK - 4) // 4, step, 0)
        jt = NCHUNK - 2
        g_wait(jt % 4, jt)
        pltpu.async_copy(bufs[jt % 4], acc_sh.at[dst_v.at[jt]],
                         ssems[jt % 4], add=True)
        jt = NCHUNK - 1
        g_wait(jt % 4, jt)
        pltpu.async_copy(bufs[jt % 4], acc_sh.at[dst_v.at[jt]],
                         ssems[jt % 4], add=True)
        for k in range(4):
            s_wait(k)
        return carry

    lax.fori_loop(0, NBLK_T // NCHUNK, chunk, 0)
    plsc.subcore_barrier()
    pltpu.sync_copy(acc_sh.at[pl.ds(s * RPT, RPT)],
                    out_h.at[c, pl.ds(s * RPT, RPT)])


@functools.cache
def _sc_scatter():
    return pl.kernel(
        _sc_body,
        out_type=jax.ShapeDtypeStruct((NC, NROW, DH), jnp.float32),
        mesh=plsc.VectorSubcoreMesh(core_axis_name="c", subcore_axis_name="s",
                                    num_cores=NC, num_subcores=NS),
        scratch_types=[
            pltpu.VMEM((NCHUNK, BLK), jnp.int32),
            pltpu.VMEM((NCHUNK, BLK), jnp.int32),
            pltpu.VMEM((BLK, DH), jnp.float32),
            pltpu.VMEM((BLK, DH), jnp.float32),
            pltpu.VMEM((BLK, DH), jnp.float32),
            pltpu.VMEM((BLK, DH), jnp.float32),
            pltpu.VMEM_SHARED((N, DH), jnp.float32),
            pltpu.VMEM_SHARED((NROW, DH), jnp.float32),
            pltpu.SemaphoreType.DMA,
            pltpu.SemaphoreType.DMA,
            pltpu.SemaphoreType.DMA,
            pltpu.SemaphoreType.DMA,
            pltpu.SemaphoreType.DMA,
            pltpu.SemaphoreType.DMA,
            pltpu.SemaphoreType.DMA,
            pltpu.SemaphoreType.DMA,
        ],
        compiler_params=pltpu.CompilerParams(use_tc_tiling_on_sc=False),
    )


def _finish_body(p_ref, w_ref, s_ref, t_ref):
    agg = jnp.concatenate([p_ref[0, :N, :], p_ref[1, :N, :]], axis=1)
    sc = lax.dot_general(agg.astype(jnp.bfloat16), w_ref[...],
                         (((1,), (1,)), ((), ())),
                         preferred_element_type=jnp.float32)
    col = lax.broadcasted_iota(jnp.int32, (N, KP), 1)
    valid = col < K1
    sm = jnp.where(valid, sc, jnp.float32(-3.4e38))
    m = jnp.max(sm, axis=1, keepdims=True)
    hit = jnp.logical_and(sm == m, valid)
    idx = jnp.where(hit, col, jnp.int32(KP))
    t_ref[...] = jnp.min(idx, axis=1, keepdims=True)
    s_ref[...] = sc[:, :K1]


def _finish(partials, w_bf):
    return pl.pallas_call(
        _finish_body,
        out_shape=(jax.ShapeDtypeStruct((N, K1), jnp.float32),
                   jax.ShapeDtypeStruct((N, 1), jnp.int32)),
    )(partials, w_bf)


def kernel(x, edge_index, W_rnn, h0, a_prelu, W_dec):
    src = edge_index[0]
    dst = edge_index[1]
    pad = E_PAD - E
    src_p = jnp.concatenate([src, jnp.zeros((pad,), jnp.int32)]).reshape(NBLK, BLK)
    dst_p = jnp.concatenate([dst, jnp.full((pad,), N, jnp.int32)]).reshape(NBLK, BLK)
    zz = jnp.zeros((BLK, DH), jnp.float32)

    def step(h, _):
        v = h @ W_rnn.T
        h_new = jnp.where(v >= 0, v, a_prelu * v)
        return h_new, h_new

    _, H = lax.scan(step, h0, None, length=K1)
    weights = H @ W_dec.T
    w_bf = jnp.concatenate(
        [weights, jnp.zeros((KP - K1, D), weights.dtype)]).astype(jnp.bfloat16)

    partials = _sc_scatter()(x, src_p, dst_p, zz)
    scores, t = _finish(partials, w_bf)
    return scores, t.reshape(N)
